# baseline (speedup 1.0000x reference)
import functools

import jax
import jax.numpy as jnp
from jax import lax
from jax.experimental import pallas as pl
from jax.experimental.pallas import tpu as pltpu
from jax.experimental.pallas import tpu_sc as plsc

N_FACTORS = 64
PAD = 128
BATCH = 16384
NUM_WORKERS = 32
B_PER_W = BATCH // NUM_WORKERS
CHUNK = 128
N_CHUNKS = B_PER_W // CHUNK
L = 16


def _bpr_body(users_ref, pos_ref, neg_ref, utab_ref, itab_ref, out_ref,
              idx_u, idx_p, idx_n, sidx_u, sidx_p, sidx_n,
              rows_u, rows_p, rows_n, out_buf, sem):
    wid = lax.axis_index("s") * 2 + lax.axis_index("c")
    row0 = wid * N_CHUNKS
    base = wid * B_PER_W

    pltpu.sync_copy(users_ref.at[pl.ds(row0, N_CHUNKS)], idx_u)
    pltpu.sync_copy(pos_ref.at[pl.ds(row0, N_CHUNKS)], idx_p)
    pltpu.sync_copy(neg_ref.at[pl.ds(row0, N_CHUNKS)], idx_n)

    for src, dst in ((idx_u, sidx_u), (idx_p, sidx_p), (idx_n, sidx_n)):
        for j in range(N_CHUNKS):
            for k in range(CHUNK // L):
                s = pl.ds(k * L, L)
                dst[j, s] = lax.shift_right_logical(src[j, s], 1)

    def fire(c):
        buf = pl.ds((c % 2) * CHUNK, CHUNK)
        return (
            pltpu.async_copy(utab_ref.at[sidx_u.at[c]], rows_u.at[buf], sem),
            pltpu.async_copy(itab_ref.at[sidx_p.at[c]], rows_p.at[buf], sem),
            pltpu.async_copy(itab_ref.at[sidx_n.at[c]], rows_n.at[buf], sem),
        )

    lanes = lax.iota(jnp.int32, L)
    handles = fire(0)
    for c in range(N_CHUNKS):
        next_handles = fire(c + 1) if c + 1 < N_CHUNKS else None
        for h in handles:
            h.wait()
        handles = next_handles

        buf_row0 = (c % 2) * CHUNK

        def group(g, _):
            gs = pl.ds(g * L, L)
            row_vec = buf_row0 + g * L + lanes
            off_u = (idx_u[c, gs] & 1) * N_FACTORS
            off_p = (idx_p[c, gs] & 1) * N_FACTORS
            off_n = (idx_n[c, gs] & 1) * N_FACTORS

            def dstep(d, acc):
                col = (d + lanes) & (N_FACTORS - 1)
                uu = plsc.load_gather(rows_u, [row_vec, off_u + col])
                pp = plsc.load_gather(rows_p, [row_vec, off_p + col])
                nn = plsc.load_gather(rows_n, [row_vec, off_n + col])
                return acc + uu * (pp - nn)

            acc = lax.fori_loop(0, N_FACTORS, dstep,
                                jnp.zeros((L,), jnp.float32), unroll=8)
            out_buf[pl.ds(c * CHUNK + g * L, L)] = acc
            return _

        lax.fori_loop(0, CHUNK // L, group, None)

    pltpu.sync_copy(out_buf, out_ref.at[pl.ds(base, B_PER_W)])


def kernel(users, pos_items, neg_items, user_table, item_table):
    users = users.astype(jnp.int32).reshape(BATCH // CHUNK, CHUNK)
    pos_items = pos_items.astype(jnp.int32).reshape(BATCH // CHUNK, CHUNK)
    neg_items = neg_items.astype(jnp.int32).reshape(BATCH // CHUNK, CHUNK)
    n_lines = user_table.shape[0] * N_FACTORS // PAD
    ut = user_table.reshape(n_lines, PAD)
    it = item_table.reshape(n_lines, PAD)

    mesh = plsc.VectorSubcoreMesh(core_axis_name="c", subcore_axis_name="s")
    run = functools.partial(
        pl.kernel,
        mesh=mesh,
        compiler_params=pltpu.CompilerParams(needs_layout_passes=False),
        out_type=jax.ShapeDtypeStruct((BATCH,), jnp.float32),
        scratch_types=[
            pltpu.VMEM((N_CHUNKS, CHUNK), jnp.int32),
            pltpu.VMEM((N_CHUNKS, CHUNK), jnp.int32),
            pltpu.VMEM((N_CHUNKS, CHUNK), jnp.int32),
            pltpu.VMEM((N_CHUNKS, CHUNK), jnp.int32),
            pltpu.VMEM((N_CHUNKS, CHUNK), jnp.int32),
            pltpu.VMEM((N_CHUNKS, CHUNK), jnp.int32),
            pltpu.VMEM((2 * CHUNK, PAD), jnp.float32),
            pltpu.VMEM((2 * CHUNK, PAD), jnp.float32),
            pltpu.VMEM((2 * CHUNK, PAD), jnp.float32),
            pltpu.VMEM((B_PER_W,), jnp.float32),
            pltpu.SemaphoreType.DMA,
        ],
    )(_bpr_body)
    return run(users, pos_items, neg_items, ut, it)
